# baseline (speedup 1.0000x reference)
import functools

import jax
import jax.numpy as jnp
from jax import lax
from jax.experimental import pallas as pl
from jax.experimental.pallas import tpu as pltpu
from jax.experimental.pallas import tpu_sc as plsc

N_AGT = 10000
N_CTX = 10000
E = 160000
D = 128

NC = 2
NS = 16
NW = NC * NS
CHUNK = 80
NCHUNK = E // CHUNK
RB = 400
BE = 4000
_EPS = 1e-5


def _gn(x, g, b):
    m = jnp.mean(x, axis=1, keepdims=True)
    xc = x - m
    v = jnp.mean(xc * xc, axis=1, keepdims=True)
    return xc * lax.rsqrt(v + _EPS) * g + b


def _pack16(a, b):
    ab = lax.bitcast_convert_type(a, jnp.int32)
    bb = lax.bitcast_convert_type(b, jnp.int32)
    return jnp.bitwise_or(jnp.bitwise_and(ab, jnp.int32(-65536)),
                          lax.shift_right_logical(bb, 16))


def _unpack16(p):
    a = lax.bitcast_convert_type(
        jnp.bitwise_and(p, jnp.int32(-65536)), jnp.float32)
    b = lax.bitcast_convert_type(lax.shift_left(p, 16), jnp.float32)
    return a, b


def _node_pre_body(agts_ref, ctx_ref, actr_ref, cctr_ref,
                    WqT, gq, bq, WBT, WCT, w1, w2,
                    ta_ref, tc_ref):
    x = agts_ref[...]
    q = jnp.dot(x, WqT[...], preferred_element_type=jnp.float32)
    q = jnp.maximum(_gn(q, gq[...], bq[...]), 0.0)
    qb = jnp.dot(q, WBT[...], preferred_element_type=jnp.float32)
    a1 = actr_ref[:, 0:1] * w1[...] + actr_ref[:, 1:2] * w2[...]
    ta_ref[...] = _pack16(qb, a1)
    cc = jnp.dot(ctx_ref[...], WCT[...], preferred_element_type=jnp.float32)
    c1 = cctr_ref[:, 0:1] * w1[...] + cctr_ref[:, 1:2] * w2[...]
    tc_ref[...] = _pack16(cc, c1)


def _node_pre(agts, ctx, agt_ctrs, ctx_ctrs, WqT, gq, bq, WBT, WCT,
              w1, w2):
    grid = (N_AGT // RB,)
    row = pl.BlockSpec((RB, D), lambda i: (i, 0))
    ctr = pl.BlockSpec((RB, 2), lambda i: (i, 0))
    full = pl.BlockSpec((D, D), lambda i: (0, 0))
    vec = pl.BlockSpec((1, D), lambda i: (0, 0))
    return pl.pallas_call(
        _node_pre_body,
        grid=grid,
        in_specs=[row, row, ctr, ctr, full, vec, vec, full, full,
                  vec, vec],
        out_specs=[row, row],
        out_shape=[jax.ShapeDtypeStruct((N_AGT, D), jnp.int32),
                   jax.ShapeDtypeStruct((N_CTX, D), jnp.int32)],
    )(agts, ctx, agt_ctrs, ctx_ctrs, WqT, gq, bq, WBT, WCT, w1, w2)


_sc_mesh = plsc.VectorSubcoreMesh(core_axis_name="c", subcore_axis_name="s",
                                  num_cores=NC, num_subcores=NS)


NBUF = 3
NBUF_D = 2
NSEG = 4
E_SEG = E // NSEG
NCHUNK_SEG = E_SEG // CHUNK
NITER = (NCHUNK_SEG + NW - 1) // NW
NITER_PAD = ((NITER + NBUF - 1) // NBUF) * NBUF
NITER_PAD_D = ((NITER + NBUF_D - 1) // NBUF_D) * NBUF_D


@functools.partial(
    pl.kernel,
    out_type=(
        jax.ShapeDtypeStruct((E_SEG, D), jnp.int32),
        jax.ShapeDtypeStruct((E_SEG, D), jnp.int32),
    ),
    mesh=_sc_mesh,
    scratch_types=[
        pltpu.VMEM((NBUF, CHUNK), jnp.int32),
        pltpu.VMEM((NBUF, CHUNK), jnp.int32),
        pltpu.VMEM((NBUF, CHUNK, D), jnp.int32),
        pltpu.VMEM((NBUF, CHUNK, D), jnp.int32),
        pltpu.SemaphoreType.DMA,
        pltpu.SemaphoreType.DMA,
        pltpu.SemaphoreType.DMA,
    ],
)
def _gather_sc(hi_hbm, wi_hbm, ta_hbm, tb_hbm,
               ga_hbm, gx_hbm,
               hi_v, wi_v, arows, xrows, sem0, sem1, sem2):
    c = lax.axis_index("c")
    s = lax.axis_index("s")
    wid = s * NC + c
    sems = (sem0, sem1, sem2)

    def start(k, b):
        cid = wid + NW * k

        @pl.when(cid < NCHUNK_SEG)
        def _():
            off = cid * CHUNK
            pltpu.sync_copy(hi_hbm.at[pl.ds(off, CHUNK)], hi_v.at[b])
            pltpu.sync_copy(wi_hbm.at[pl.ds(off, CHUNK)], wi_v.at[b])
            pltpu.async_copy(ta_hbm.at[hi_v.at[b]], arows.at[b], sems[b])
            pltpu.async_copy(tb_hbm.at[wi_v.at[b]], xrows.at[b], sems[b])

    def drain_and_flush(k, b):
        cid = wid + NW * k

        @pl.when(cid < NCHUNK_SEG)
        def _():
            dummy = ta_hbm.at[pl.ds(0, CHUNK)]
            pltpu.make_async_copy(dummy, arows.at[b], sems[b]).wait()
            pltpu.make_async_copy(dummy, xrows.at[b], sems[b]).wait()
            off = cid * CHUNK
            pltpu.sync_copy(arows.at[b], ga_hbm.at[pl.ds(off, CHUNK)])
            pltpu.sync_copy(xrows.at[b], gx_hbm.at[pl.ds(off, CHUNK)])

    for b in range(NBUF):
        start(b, b)

    @pl.loop(0, NITER_PAD, step=NBUF)
    def outer(k):
        for b in range(NBUF):
            drain_and_flush(k + b, b)
            start(k + b + NBUF, b)


def _edge_mlp_body(ga_ref, gx_ref,
                   bd1, Wd2T, gd2, bd2, AT, gc1, bc1, u_ref):
    qa, a1 = _unpack16(ga_ref[...])
    cc, c1 = _unpack16(gx_ref[...])
    e1 = jnp.maximum(a1 - c1 + bd1[...], 0.0)
    e2 = jnp.dot(e1, Wd2T[...], preferred_element_type=jnp.float32)
    e2 = jnp.maximum(_gn(e2, gd2[...], bd2[...]), 0.0)
    y = (jnp.dot(e2, AT[...], preferred_element_type=jnp.float32)
         + qa + cc)
    u_ref[...] = jnp.maximum(_gn(y, gc1[...], bc1[...]), 0.0)


def _edge_mlp(Ga, Gx, bd1, Wd2T, gd2, bd2, AT, gc1, bc1):
    grid = (E_SEG // BE,)
    row2 = pl.BlockSpec((BE, D), lambda i: (i, 0))
    row = pl.BlockSpec((BE, D), lambda i: (i, 0))
    full = pl.BlockSpec((D, D), lambda i: (0, 0))
    vec = pl.BlockSpec((1, D), lambda i: (0, 0))
    return pl.pallas_call(
        _edge_mlp_body,
        grid=grid,
        in_specs=[row2, row2, vec, full, vec, vec, full, vec, vec],
        out_specs=row,
        out_shape=jax.ShapeDtypeStruct((E_SEG, D), jnp.float32),
    )(Ga, Gx, bd1, Wd2T, gd2, bd2, AT, gc1, bc1)


ZR = 48
RSUB = 624
TAIL = N_AGT - NS * RSUB
CH_PER_CORE = NCHUNK // NC


@functools.partial(
    pl.kernel,
    out_type=jax.ShapeDtypeStruct((NC, N_AGT, D), jnp.float32),
    mesh=_sc_mesh,
    scratch_types=[
        pltpu.VMEM((NBUF_D, CHUNK), jnp.int32),
        pltpu.VMEM((NBUF_D, CHUNK, D), jnp.float32),
        pltpu.VMEM((ZR, D), jnp.float32),
        pltpu.VMEM_SHARED((N_AGT, D), jnp.float32),
        pltpu.SemaphoreType.DMA,
        pltpu.SemaphoreType.DMA,
    ],
)
def _scatter_sc(u_hbm, hi_hbm, p_hbm, hi_v, rows, zbuf, acc_sh,
                sem0, sem1):
    c = lax.axis_index("c")
    s = lax.axis_index("s")
    wid = s * NC + c
    sems = (sem0, sem1)
    zero16 = jnp.zeros((16,), jnp.float32)
    for r in range(ZR):
        for j in range(D // 16):
            zbuf[r, pl.ds(j * 16, 16)] = zero16
    for t in range(RSUB // ZR):
        pltpu.sync_copy(zbuf, acc_sh.at[pl.ds(s * RSUB + t * ZR, ZR)])

    @pl.when(s == NS - 1)
    def _():
        pltpu.sync_copy(zbuf.at[pl.ds(0, TAIL)],
                        acc_sh.at[pl.ds(NS * RSUB, TAIL)])

    plsc.subcore_barrier()

    def start(k, b):
        cid = wid + NW * k

        @pl.when(cid < NCHUNK_SEG)
        def _():
            off = cid * CHUNK
            pltpu.sync_copy(hi_hbm.at[pl.ds(off, CHUNK)], hi_v.at[b])
            pltpu.async_copy(u_hbm.at[pl.ds(off, CHUNK)], rows.at[b],
                             sems[b])

    def drain_and_add(k, b):
        cid = wid + NW * k

        @pl.when(cid < NCHUNK_SEG)
        def _():
            pltpu.make_async_copy(u_hbm.at[pl.ds(0, CHUNK)], rows.at[b],
                                  sems[b]).wait()
            pltpu.sync_copy(rows.at[b], acc_sh.at[hi_v.at[b]], add=True)

    for b in range(NBUF_D):
        start(b, b)

    @pl.loop(0, NITER_PAD_D, step=NBUF_D)
    def outer(k):
        for b in range(NBUF_D):
            drain_and_add(k + b, b)
            start(k + b + NBUF_D, b)

    plsc.subcore_barrier()
    pltpu.sync_copy(acc_sh.at[pl.ds(s * RSUB, RSUB)],
                    p_hbm.at[c, pl.ds(s * RSUB, RSUB)])

    @pl.when(s == NS - 1)
    def _():
        pltpu.sync_copy(acc_sh.at[pl.ds(NS * RSUB, TAIL)],
                        p_hbm.at[c, pl.ds(NS * RSUB, TAIL)])


def _final_body(*refs):
    p_refs = refs[:NSEG]
    (agts_ref, WaT, Wc2T, gn_, bn_, WlT, gl_, bl_, out_ref) = refs[NSEG:]
    x = agts_ref[...]
    u = p_refs[0][0] + p_refs[0][1]
    for pr in p_refs[1:]:
        u = u + pr[0] + pr[1]
    out = (jnp.dot(x, WaT[...], preferred_element_type=jnp.float32)
           + jnp.dot(u, Wc2T[...], preferred_element_type=jnp.float32))
    out = jnp.maximum(_gn(out, gn_[...], bn_[...]), 0.0)
    out = _gn(jnp.dot(out, WlT[...], preferred_element_type=jnp.float32),
              gl_[...], bl_[...])
    out_ref[...] = jnp.maximum(out + x, 0.0)


def _final(parts, agts, WaT, Wc2T, gn_, bn_, WlT, gl_, bl_):
    grid = (N_AGT // RB,)
    row = pl.BlockSpec((RB, D), lambda i: (i, 0))
    prow = pl.BlockSpec((NC, RB, D), lambda i: (0, i, 0))
    full = pl.BlockSpec((D, D), lambda i: (0, 0))
    vec = pl.BlockSpec((1, D), lambda i: (0, 0))
    return pl.pallas_call(
        _final_body,
        grid=grid,
        in_specs=[prow] * NSEG + [row, full, full, vec, vec, full, vec, vec],
        out_specs=row,
        out_shape=jax.ShapeDtypeStruct((N_AGT, D), jnp.float32),
    )(*parts, agts, WaT, Wc2T, gn_, bn_, WlT, gl_, bl_)


def kernel(agts, agt_ctrs, ctx, ctx_ctrs, hi, wi,
           W_d1, b_d1, W_d2, g_d2, b_d2,
           W_q, g_q, b_q,
           W_c1, g_c1, b_c1, W_c2,
           W_a, g_n, b_n,
           W_l, g_l, b_l):
    AT = W_c1[:, :D].T
    BT = W_c1[:, D:2 * D].T
    CT = W_c1[:, 2 * D:].T
    r = lambda v: v.reshape(1, D)

    TA, TB = _node_pre(agts, ctx, agt_ctrs, ctx_ctrs,
                       W_q.T, r(g_q), r(b_q), BT, CT,
                       r(W_d1[:, 0]), r(W_d1[:, 1]))

    parts = []
    for g in range(NSEG):
        hi_s = lax.slice_in_dim(hi, g * E_SEG, (g + 1) * E_SEG)
        wi_s = lax.slice_in_dim(wi, g * E_SEG, (g + 1) * E_SEG)
        Ga, Gx = _gather_sc(hi_s, wi_s, TA, TB)
        u = _edge_mlp(Ga, Gx, r(b_d1),
                      W_d2.T, r(g_d2), r(b_d2), AT, r(g_c1), r(b_c1))
        parts.append(_scatter_sc(u, hi_s))

    return _final(parts, agts, W_a.T, W_c2.T,
                  r(g_n), r(b_n), W_l.T, r(g_l), r(b_l))


# speedup vs baseline: 1.1860x; 1.1860x over previous
import functools

import jax
import jax.numpy as jnp
from jax import lax
from jax.experimental import pallas as pl
from jax.experimental.pallas import tpu as pltpu
from jax.experimental.pallas import tpu_sc as plsc

N_AGT = 10000
N_CTX = 10000
E = 160000
D = 128

NC = 2
NS = 16
NW = NC * NS
CHUNK = 128
NCHUNK = E // CHUNK
RB = 400
BE = 3200
_EPS = 1e-5


def _gn(x, g, b):
    m = jnp.mean(x, axis=1, keepdims=True)
    xc = x - m
    v = jnp.mean(xc * xc, axis=1, keepdims=True)
    return xc * lax.rsqrt(v + _EPS) * g + b


def _pack16(a, b):
    ab = lax.bitcast_convert_type(a, jnp.int32)
    bb = lax.bitcast_convert_type(b, jnp.int32)
    return jnp.bitwise_or(jnp.bitwise_and(ab, jnp.int32(-65536)),
                          lax.shift_right_logical(bb, 16))


def _unpack16(p):
    a = lax.bitcast_convert_type(
        jnp.bitwise_and(p, jnp.int32(-65536)), jnp.float32)
    b = lax.bitcast_convert_type(lax.shift_left(p, 16), jnp.float32)
    return a, b


def _node_pre_body(agts_ref, ctx_ref, actr_ref, cctr_ref,
                    WqT, gq, bq, WBT, WCT, w1, w2,
                    ta_ref, tc_ref):
    x = agts_ref[...]
    q = jnp.dot(x, WqT[...], preferred_element_type=jnp.float32)
    q = jnp.maximum(_gn(q, gq[...], bq[...]), 0.0)
    qb = jnp.dot(q, WBT[...], preferred_element_type=jnp.float32)
    a1 = actr_ref[:, 0:1] * w1[...] + actr_ref[:, 1:2] * w2[...]
    ta_ref[...] = _pack16(qb, a1)
    cc = jnp.dot(ctx_ref[...], WCT[...], preferred_element_type=jnp.float32)
    c1 = cctr_ref[:, 0:1] * w1[...] + cctr_ref[:, 1:2] * w2[...]
    tc_ref[...] = _pack16(cc, c1)


def _node_pre(agts, ctx, agt_ctrs, ctx_ctrs, WqT, gq, bq, WBT, WCT,
              w1, w2):
    grid = (N_AGT // RB,)
    row = pl.BlockSpec((RB, D), lambda i: (i, 0))
    ctr = pl.BlockSpec((RB, 2), lambda i: (i, 0))
    full = pl.BlockSpec((D, D), lambda i: (0, 0))
    vec = pl.BlockSpec((1, D), lambda i: (0, 0))
    return pl.pallas_call(
        _node_pre_body,
        grid=grid,
        in_specs=[row, row, ctr, ctr, full, vec, vec, full, full,
                  vec, vec],
        out_specs=[row, row],
        out_shape=[jax.ShapeDtypeStruct((N_AGT, D), jnp.int32),
                   jax.ShapeDtypeStruct((N_CTX, D), jnp.int32)],
    )(agts, ctx, agt_ctrs, ctx_ctrs, WqT, gq, bq, WBT, WCT, w1, w2)


_sc_mesh = plsc.VectorSubcoreMesh(core_axis_name="c", subcore_axis_name="s",
                                  num_cores=NC, num_subcores=NS)


NBUF = 3
NBUF_D = 2
NSEG = 2
E_SEG = E // NSEG
NCHUNK_SEG = E_SEG // CHUNK
NITER = (NCHUNK_SEG + NW - 1) // NW
NITER_PAD = ((NITER + NBUF - 1) // NBUF) * NBUF
NITER_PAD_D = ((NITER + NBUF_D - 1) // NBUF_D) * NBUF_D


@functools.partial(
    pl.kernel,
    out_type=(
        jax.ShapeDtypeStruct((E_SEG, D), jnp.int32),
        jax.ShapeDtypeStruct((E_SEG, D), jnp.int32),
    ),
    mesh=_sc_mesh,
    scratch_types=[
        pltpu.VMEM((NBUF, CHUNK), jnp.int32),
        pltpu.VMEM((NBUF, CHUNK), jnp.int32),
        pltpu.VMEM((NBUF, CHUNK, D), jnp.int32),
        pltpu.VMEM((NBUF, CHUNK, D), jnp.int32),
        pltpu.SemaphoreType.DMA,
        pltpu.SemaphoreType.DMA,
        pltpu.SemaphoreType.DMA,
    ],
)
def _gather_sc(hi_hbm, wi_hbm, ta_hbm, tb_hbm,
               ga_hbm, gx_hbm,
               hi_v, wi_v, arows, xrows, sem0, sem1, sem2):
    c = lax.axis_index("c")
    s = lax.axis_index("s")
    wid = s * NC + c
    sems = (sem0, sem1, sem2)

    def start(k, b):
        cid = wid + NW * k

        @pl.when(cid < NCHUNK_SEG)
        def _():
            off = cid * CHUNK
            pltpu.sync_copy(hi_hbm.at[pl.ds(off, CHUNK)], hi_v.at[b])
            pltpu.sync_copy(wi_hbm.at[pl.ds(off, CHUNK)], wi_v.at[b])
            pltpu.async_copy(ta_hbm.at[hi_v.at[b]], arows.at[b], sems[b])
            pltpu.async_copy(tb_hbm.at[wi_v.at[b]], xrows.at[b], sems[b])

    def drain_and_flush(k, b):
        cid = wid + NW * k

        @pl.when(cid < NCHUNK_SEG)
        def _():
            dummy = ta_hbm.at[pl.ds(0, CHUNK)]
            pltpu.make_async_copy(dummy, arows.at[b], sems[b]).wait()
            pltpu.make_async_copy(dummy, xrows.at[b], sems[b]).wait()
            off = cid * CHUNK
            pltpu.sync_copy(arows.at[b], ga_hbm.at[pl.ds(off, CHUNK)])
            pltpu.sync_copy(xrows.at[b], gx_hbm.at[pl.ds(off, CHUNK)])

    for b in range(NBUF):
        start(b, b)

    @pl.loop(0, NITER_PAD, step=NBUF)
    def outer(k):
        for b in range(NBUF):
            drain_and_flush(k + b, b)
            start(k + b + NBUF, b)


def _edge_mlp_body(ga_ref, gx_ref,
                   bd1, Wd2T, gd2, bd2, AT, gc1, bc1, u_ref):
    qa, a1 = _unpack16(ga_ref[...])
    cc, c1 = _unpack16(gx_ref[...])
    e1 = jnp.maximum(a1 - c1 + bd1[...], 0.0)
    e2 = jnp.dot(e1, Wd2T[...], preferred_element_type=jnp.float32)
    e2 = jnp.maximum(_gn(e2, gd2[...], bd2[...]), 0.0)
    y = (jnp.dot(e2, AT[...], preferred_element_type=jnp.float32)
         + qa + cc)
    u_ref[...] = jnp.maximum(_gn(y, gc1[...], bc1[...]), 0.0)


def _edge_mlp(Ga, Gx, bd1, Wd2T, gd2, bd2, AT, gc1, bc1):
    grid = (E_SEG // BE,)
    row2 = pl.BlockSpec((BE, D), lambda i: (i, 0))
    row = pl.BlockSpec((BE, D), lambda i: (i, 0))
    full = pl.BlockSpec((D, D), lambda i: (0, 0))
    vec = pl.BlockSpec((1, D), lambda i: (0, 0))
    return pl.pallas_call(
        _edge_mlp_body,
        grid=grid,
        in_specs=[row2, row2, vec, full, vec, vec, full, vec, vec],
        out_specs=row,
        out_shape=jax.ShapeDtypeStruct((E_SEG, D), jnp.float32),
    )(Ga, Gx, bd1, Wd2T, gd2, bd2, AT, gc1, bc1)


ZR = 48
RSUB = 624
TAIL = N_AGT - NS * RSUB
CH_PER_CORE = NCHUNK // NC


@functools.partial(
    pl.kernel,
    out_type=jax.ShapeDtypeStruct((NC, N_AGT, D), jnp.float32),
    mesh=_sc_mesh,
    scratch_types=[
        pltpu.VMEM((NBUF_D, CHUNK), jnp.int32),
        pltpu.VMEM((NBUF_D, CHUNK, D), jnp.float32),
        pltpu.VMEM((ZR, D), jnp.float32),
        pltpu.VMEM_SHARED((N_AGT, D), jnp.float32),
        pltpu.SemaphoreType.DMA,
        pltpu.SemaphoreType.DMA,
    ],
)
def _scatter_sc(u_hbm, hi_hbm, p_hbm, hi_v, rows, zbuf, acc_sh,
                sem0, sem1):
    c = lax.axis_index("c")
    s = lax.axis_index("s")
    wid = s * NC + c
    sems = (sem0, sem1)
    zero16 = jnp.zeros((16,), jnp.float32)
    for r in range(ZR):
        for j in range(D // 16):
            zbuf[r, pl.ds(j * 16, 16)] = zero16
    for t in range(RSUB // ZR):
        pltpu.sync_copy(zbuf, acc_sh.at[pl.ds(s * RSUB + t * ZR, ZR)])

    @pl.when(s == NS - 1)
    def _():
        pltpu.sync_copy(zbuf.at[pl.ds(0, TAIL)],
                        acc_sh.at[pl.ds(NS * RSUB, TAIL)])

    plsc.subcore_barrier()

    def start(k, b):
        cid = wid + NW * k

        @pl.when(cid < NCHUNK_SEG)
        def _():
            off = cid * CHUNK
            pltpu.sync_copy(hi_hbm.at[pl.ds(off, CHUNK)], hi_v.at[b])
            pltpu.async_copy(u_hbm.at[pl.ds(off, CHUNK)], rows.at[b],
                             sems[b])

    def drain_and_add(k, b):
        cid = wid + NW * k

        @pl.when(cid < NCHUNK_SEG)
        def _():
            pltpu.make_async_copy(u_hbm.at[pl.ds(0, CHUNK)], rows.at[b],
                                  sems[b]).wait()
            pltpu.sync_copy(rows.at[b], acc_sh.at[hi_v.at[b]], add=True)

    for b in range(NBUF_D):
        start(b, b)

    @pl.loop(0, NITER_PAD_D, step=NBUF_D)
    def outer(k):
        for b in range(NBUF_D):
            drain_and_add(k + b, b)
            start(k + b + NBUF_D, b)

    plsc.subcore_barrier()
    pltpu.sync_copy(acc_sh.at[pl.ds(s * RSUB, RSUB)],
                    p_hbm.at[c, pl.ds(s * RSUB, RSUB)])

    @pl.when(s == NS - 1)
    def _():
        pltpu.sync_copy(acc_sh.at[pl.ds(NS * RSUB, TAIL)],
                        p_hbm.at[c, pl.ds(NS * RSUB, TAIL)])


def _final_body(*refs):
    p_refs = refs[:NSEG]
    (agts_ref, WaT, Wc2T, gn_, bn_, WlT, gl_, bl_, out_ref) = refs[NSEG:]
    x = agts_ref[...]
    u = p_refs[0][0] + p_refs[0][1]
    for pr in p_refs[1:]:
        u = u + pr[0] + pr[1]
    out = (jnp.dot(x, WaT[...], preferred_element_type=jnp.float32)
           + jnp.dot(u, Wc2T[...], preferred_element_type=jnp.float32))
    out = jnp.maximum(_gn(out, gn_[...], bn_[...]), 0.0)
    out = _gn(jnp.dot(out, WlT[...], preferred_element_type=jnp.float32),
              gl_[...], bl_[...])
    out_ref[...] = jnp.maximum(out + x, 0.0)


def _final(parts, agts, WaT, Wc2T, gn_, bn_, WlT, gl_, bl_):
    grid = (N_AGT // RB,)
    row = pl.BlockSpec((RB, D), lambda i: (i, 0))
    prow = pl.BlockSpec((NC, RB, D), lambda i: (0, i, 0))
    full = pl.BlockSpec((D, D), lambda i: (0, 0))
    vec = pl.BlockSpec((1, D), lambda i: (0, 0))
    return pl.pallas_call(
        _final_body,
        grid=grid,
        in_specs=[prow] * NSEG + [row, full, full, vec, vec, full, vec, vec],
        out_specs=row,
        out_shape=jax.ShapeDtypeStruct((N_AGT, D), jnp.float32),
    )(*parts, agts, WaT, Wc2T, gn_, bn_, WlT, gl_, bl_)


def kernel(agts, agt_ctrs, ctx, ctx_ctrs, hi, wi,
           W_d1, b_d1, W_d2, g_d2, b_d2,
           W_q, g_q, b_q,
           W_c1, g_c1, b_c1, W_c2,
           W_a, g_n, b_n,
           W_l, g_l, b_l):
    AT = W_c1[:, :D].T
    BT = W_c1[:, D:2 * D].T
    CT = W_c1[:, 2 * D:].T
    r = lambda v: v.reshape(1, D)

    TA, TB = _node_pre(agts, ctx, agt_ctrs, ctx_ctrs,
                       W_q.T, r(g_q), r(b_q), BT, CT,
                       r(W_d1[:, 0]), r(W_d1[:, 1]))

    parts = []
    for g in range(NSEG):
        hi_s = lax.slice_in_dim(hi, g * E_SEG, (g + 1) * E_SEG)
        wi_s = lax.slice_in_dim(wi, g * E_SEG, (g + 1) * E_SEG)
        Ga, Gx = _gather_sc(hi_s, wi_s, TA, TB)
        u = _edge_mlp(Ga, Gx, r(b_d1),
                      W_d2.T, r(g_d2), r(b_d2), AT, r(g_c1), r(b_c1))
        parts.append(_scatter_sc(u, hi_s))

    return _final(parts, agts, W_a.T, W_c2.T,
                  r(g_n), r(b_n), W_l.T, r(g_l), r(b_l))


# speedup vs baseline: 1.2699x; 1.0707x over previous
import functools

import jax
import jax.numpy as jnp
from jax import lax
from jax.experimental import pallas as pl
from jax.experimental.pallas import tpu as pltpu
from jax.experimental.pallas import tpu_sc as plsc

N_AGT = 10000
N_CTX = 10000
E = 160000
D = 128

NC = 2
NS = 16
NW = NC * NS
CHUNK = 128
NCHUNK = E // CHUNK
RB = 1000
BE = 5000
_EPS = 1e-5


def _gn(x, g, b):
    m = jnp.mean(x, axis=1, keepdims=True)
    xc = x - m
    v = jnp.mean(xc * xc, axis=1, keepdims=True)
    return xc * lax.rsqrt(v + _EPS) * g + b


def _pack16(a, b):
    ab = lax.bitcast_convert_type(a, jnp.int32)
    bb = lax.bitcast_convert_type(b, jnp.int32)
    return jnp.bitwise_or(jnp.bitwise_and(ab, jnp.int32(-65536)),
                          lax.shift_right_logical(bb, 16))


def _unpack16(p):
    a = lax.bitcast_convert_type(
        jnp.bitwise_and(p, jnp.int32(-65536)), jnp.float32)
    b = lax.bitcast_convert_type(lax.shift_left(p, 16), jnp.float32)
    return a, b


def _node_pre_body(agts_ref, ctx_ref, actr_ref, cctr_ref,
                    WqT, gq, bq, WBT, WCT, w1, w2,
                    ta_ref, tc_ref):
    x = agts_ref[...]
    q = jnp.dot(x, WqT[...], preferred_element_type=jnp.float32)
    q = jnp.maximum(_gn(q, gq[...], bq[...]), 0.0)
    qb = jnp.dot(q, WBT[...], preferred_element_type=jnp.float32)
    a1 = actr_ref[:, 0:1] * w1[...] + actr_ref[:, 1:2] * w2[...]
    ta_ref[...] = _pack16(qb, a1)
    cc = jnp.dot(ctx_ref[...], WCT[...], preferred_element_type=jnp.float32)
    c1 = cctr_ref[:, 0:1] * w1[...] + cctr_ref[:, 1:2] * w2[...]
    tc_ref[...] = _pack16(cc, c1)


def _node_pre(agts, ctx, agt_ctrs, ctx_ctrs, WqT, gq, bq, WBT, WCT,
              w1, w2):
    grid = (N_AGT // RB,)
    row = pl.BlockSpec((RB, D), lambda i: (i, 0))
    ctr = pl.BlockSpec((RB, 2), lambda i: (i, 0))
    full = pl.BlockSpec((D, D), lambda i: (0, 0))
    vec = pl.BlockSpec((1, D), lambda i: (0, 0))
    return pl.pallas_call(
        _node_pre_body,
        grid=grid,
        in_specs=[row, row, ctr, ctr, full, vec, vec, full, full,
                  vec, vec],
        out_specs=[row, row],
        out_shape=[jax.ShapeDtypeStruct((N_AGT, D), jnp.int32),
                   jax.ShapeDtypeStruct((N_CTX, D), jnp.int32)],
    )(agts, ctx, agt_ctrs, ctx_ctrs, WqT, gq, bq, WBT, WCT, w1, w2)


_sc_mesh = plsc.VectorSubcoreMesh(core_axis_name="c", subcore_axis_name="s",
                                  num_cores=NC, num_subcores=NS)


NBUF = 3
NBUF_D = 2
NSEG = 2
E_SEG = E // NSEG
NCHUNK_SEG = E_SEG // CHUNK
NITER = (NCHUNK_SEG + NW - 1) // NW
NITER_PAD = ((NITER + NBUF - 1) // NBUF) * NBUF
NITER_PAD_D = ((NITER + NBUF_D - 1) // NBUF_D) * NBUF_D


@functools.partial(
    pl.kernel,
    out_type=(
        jax.ShapeDtypeStruct((E_SEG, D), jnp.int32),
        jax.ShapeDtypeStruct((E_SEG, D), jnp.int32),
    ),
    mesh=_sc_mesh,
    scratch_types=[
        pltpu.VMEM((NBUF, CHUNK), jnp.int32),
        pltpu.VMEM((NBUF, CHUNK), jnp.int32),
        pltpu.VMEM((NBUF, CHUNK, D), jnp.int32),
        pltpu.VMEM((NBUF, CHUNK, D), jnp.int32),
        pltpu.SemaphoreType.DMA,
        pltpu.SemaphoreType.DMA,
        pltpu.SemaphoreType.DMA,
    ],
)
def _gather_sc(hi_hbm, wi_hbm, ta_hbm, tb_hbm,
               ga_hbm, gx_hbm,
               hi_v, wi_v, arows, xrows, sem0, sem1, sem2):
    c = lax.axis_index("c")
    s = lax.axis_index("s")
    wid = s * NC + c
    sems = (sem0, sem1, sem2)

    def start(k, b):
        cid = wid + NW * k

        @pl.when(cid < NCHUNK_SEG)
        def _():
            off = cid * CHUNK
            pltpu.sync_copy(hi_hbm.at[pl.ds(off, CHUNK)], hi_v.at[b])
            pltpu.sync_copy(wi_hbm.at[pl.ds(off, CHUNK)], wi_v.at[b])
            pltpu.async_copy(ta_hbm.at[hi_v.at[b]], arows.at[b], sems[b])
            pltpu.async_copy(tb_hbm.at[wi_v.at[b]], xrows.at[b], sems[b])

    def drain_and_flush(k, b):
        cid = wid + NW * k

        @pl.when(cid < NCHUNK_SEG)
        def _():
            dummy = ta_hbm.at[pl.ds(0, CHUNK)]
            pltpu.make_async_copy(dummy, arows.at[b], sems[b]).wait()
            pltpu.make_async_copy(dummy, xrows.at[b], sems[b]).wait()
            off = cid * CHUNK
            pltpu.sync_copy(arows.at[b], ga_hbm.at[pl.ds(off, CHUNK)])
            pltpu.sync_copy(xrows.at[b], gx_hbm.at[pl.ds(off, CHUNK)])

    for b in range(NBUF):
        start(b, b)

    @pl.loop(0, NITER_PAD, step=NBUF)
    def outer(k):
        for b in range(NBUF):
            drain_and_flush(k + b, b)
            start(k + b + NBUF, b)


def _edge_mlp_body(ga_ref, gx_ref,
                   bd1, Wd2T, gd2, bd2, AT, gc1, bc1, u_ref):
    qa, a1 = _unpack16(ga_ref[...])
    cc, c1 = _unpack16(gx_ref[...])
    e1 = jnp.maximum(a1 - c1 + bd1[...], 0.0)
    e2 = jnp.dot(e1, Wd2T[...], preferred_element_type=jnp.float32)
    e2 = jnp.maximum(_gn(e2, gd2[...], bd2[...]), 0.0)
    y = (jnp.dot(e2, AT[...], preferred_element_type=jnp.float32)
         + qa + cc)
    u_ref[...] = jnp.maximum(_gn(y, gc1[...], bc1[...]), 0.0)


def _edge_mlp(Ga, Gx, bd1, Wd2T, gd2, bd2, AT, gc1, bc1):
    grid = (E_SEG // BE,)
    row2 = pl.BlockSpec((BE, D), lambda i: (i, 0))
    row = pl.BlockSpec((BE, D), lambda i: (i, 0))
    full = pl.BlockSpec((D, D), lambda i: (0, 0))
    vec = pl.BlockSpec((1, D), lambda i: (0, 0))
    return pl.pallas_call(
        _edge_mlp_body,
        grid=grid,
        in_specs=[row2, row2, vec, full, vec, vec, full, vec, vec],
        out_specs=row,
        out_shape=jax.ShapeDtypeStruct((E_SEG, D), jnp.float32),
    )(Ga, Gx, bd1, Wd2T, gd2, bd2, AT, gc1, bc1)


ZR = 48
RSUB = 624
TAIL = N_AGT - NS * RSUB
CH_PER_CORE = NCHUNK // NC


@functools.partial(
    pl.kernel,
    out_type=jax.ShapeDtypeStruct((NC, N_AGT, D), jnp.float32),
    mesh=_sc_mesh,
    scratch_types=[
        pltpu.VMEM((NBUF_D, CHUNK), jnp.int32),
        pltpu.VMEM((NBUF_D, CHUNK, D), jnp.float32),
        pltpu.VMEM((ZR, D), jnp.float32),
        pltpu.VMEM_SHARED((N_AGT, D), jnp.float32),
        pltpu.SemaphoreType.DMA,
        pltpu.SemaphoreType.DMA,
    ],
)
def _scatter_sc(u_hbm, hi_hbm, p_hbm, hi_v, rows, zbuf, acc_sh,
                sem0, sem1):
    c = lax.axis_index("c")
    s = lax.axis_index("s")
    wid = s * NC + c
    sems = (sem0, sem1)
    zero16 = jnp.zeros((16,), jnp.float32)
    for r in range(ZR):
        for j in range(D // 16):
            zbuf[r, pl.ds(j * 16, 16)] = zero16
    for t in range(RSUB // ZR):
        pltpu.sync_copy(zbuf, acc_sh.at[pl.ds(s * RSUB + t * ZR, ZR)])

    @pl.when(s == NS - 1)
    def _():
        pltpu.sync_copy(zbuf.at[pl.ds(0, TAIL)],
                        acc_sh.at[pl.ds(NS * RSUB, TAIL)])

    plsc.subcore_barrier()

    def start(k, b):
        cid = wid + NW * k

        @pl.when(cid < NCHUNK_SEG)
        def _():
            off = cid * CHUNK
            pltpu.sync_copy(hi_hbm.at[pl.ds(off, CHUNK)], hi_v.at[b])
            pltpu.async_copy(u_hbm.at[pl.ds(off, CHUNK)], rows.at[b],
                             sems[b])

    def drain_and_add(k, b):
        cid = wid + NW * k

        @pl.when(cid < NCHUNK_SEG)
        def _():
            pltpu.make_async_copy(u_hbm.at[pl.ds(0, CHUNK)], rows.at[b],
                                  sems[b]).wait()
            pltpu.sync_copy(rows.at[b], acc_sh.at[hi_v.at[b]], add=True)

    for b in range(NBUF_D):
        start(b, b)

    @pl.loop(0, NITER_PAD_D, step=NBUF_D)
    def outer(k):
        for b in range(NBUF_D):
            drain_and_add(k + b, b)
            start(k + b + NBUF_D, b)

    plsc.subcore_barrier()
    pltpu.sync_copy(acc_sh.at[pl.ds(s * RSUB, RSUB)],
                    p_hbm.at[c, pl.ds(s * RSUB, RSUB)])

    @pl.when(s == NS - 1)
    def _():
        pltpu.sync_copy(acc_sh.at[pl.ds(NS * RSUB, TAIL)],
                        p_hbm.at[c, pl.ds(NS * RSUB, TAIL)])


def _final_body(*refs):
    p_refs = refs[:NSEG]
    (agts_ref, WaT, Wc2T, gn_, bn_, WlT, gl_, bl_, out_ref) = refs[NSEG:]
    x = agts_ref[...]
    u = p_refs[0][0] + p_refs[0][1]
    for pr in p_refs[1:]:
        u = u + pr[0] + pr[1]
    out = (jnp.dot(x, WaT[...], preferred_element_type=jnp.float32)
           + jnp.dot(u, Wc2T[...], preferred_element_type=jnp.float32))
    out = jnp.maximum(_gn(out, gn_[...], bn_[...]), 0.0)
    out = _gn(jnp.dot(out, WlT[...], preferred_element_type=jnp.float32),
              gl_[...], bl_[...])
    out_ref[...] = jnp.maximum(out + x, 0.0)


def _final(parts, agts, WaT, Wc2T, gn_, bn_, WlT, gl_, bl_):
    grid = (N_AGT // RB,)
    row = pl.BlockSpec((RB, D), lambda i: (i, 0))
    prow = pl.BlockSpec((NC, RB, D), lambda i: (0, i, 0))
    full = pl.BlockSpec((D, D), lambda i: (0, 0))
    vec = pl.BlockSpec((1, D), lambda i: (0, 0))
    return pl.pallas_call(
        _final_body,
        grid=grid,
        in_specs=[prow] * NSEG + [row, full, full, vec, vec, full, vec, vec],
        out_specs=row,
        out_shape=jax.ShapeDtypeStruct((N_AGT, D), jnp.float32),
    )(*parts, agts, WaT, Wc2T, gn_, bn_, WlT, gl_, bl_)


def kernel(agts, agt_ctrs, ctx, ctx_ctrs, hi, wi,
           W_d1, b_d1, W_d2, g_d2, b_d2,
           W_q, g_q, b_q,
           W_c1, g_c1, b_c1, W_c2,
           W_a, g_n, b_n,
           W_l, g_l, b_l):
    AT = W_c1[:, :D].T
    BT = W_c1[:, D:2 * D].T
    CT = W_c1[:, 2 * D:].T
    r = lambda v: v.reshape(1, D)

    TA, TB = _node_pre(agts, ctx, agt_ctrs, ctx_ctrs,
                       W_q.T, r(g_q), r(b_q), BT, CT,
                       r(W_d1[:, 0]), r(W_d1[:, 1]))

    parts = []
    for g in range(NSEG):
        hi_s = lax.slice_in_dim(hi, g * E_SEG, (g + 1) * E_SEG)
        wi_s = lax.slice_in_dim(wi, g * E_SEG, (g + 1) * E_SEG)
        Ga, Gx = _gather_sc(hi_s, wi_s, TA, TB)
        u = _edge_mlp(Ga, Gx, r(b_d1),
                      W_d2.T, r(g_d2), r(b_d2), AT, r(g_c1), r(b_c1))
        parts.append(_scatter_sc(u, hi_s))

    return _final(parts, agts, W_a.T, W_c2.T,
                  r(g_n), r(b_n), W_l.T, r(g_l), r(b_l))


# speedup vs baseline: 1.2946x; 1.0194x over previous
import functools

import jax
import jax.numpy as jnp
from jax import lax
from jax.experimental import pallas as pl
from jax.experimental.pallas import tpu as pltpu
from jax.experimental.pallas import tpu_sc as plsc

N_AGT = 10000
N_CTX = 10000
E = 160000
D = 128

NC = 2
NS = 16
NW = NC * NS
CHUNK = 128
NCHUNK = E // CHUNK
RB = 2000
BE = 8000
_EPS = 1e-5


def _gn(x, g, b):
    m = jnp.mean(x, axis=1, keepdims=True)
    xc = x - m
    v = jnp.mean(xc * xc, axis=1, keepdims=True)
    return xc * lax.rsqrt(v + _EPS) * g + b


def _pack16(a, b):
    ab = lax.bitcast_convert_type(a, jnp.int32)
    bb = lax.bitcast_convert_type(b, jnp.int32)
    return jnp.bitwise_or(jnp.bitwise_and(ab, jnp.int32(-65536)),
                          lax.shift_right_logical(bb, 16))


def _unpack16(p):
    a = lax.bitcast_convert_type(
        jnp.bitwise_and(p, jnp.int32(-65536)), jnp.float32)
    b = lax.bitcast_convert_type(lax.shift_left(p, 16), jnp.float32)
    return a, b


def _node_pre_body(agts_ref, ctx_ref, actr_ref, cctr_ref,
                    WqT, gq, bq, WBT, WCT, w1, w2,
                    ta_ref, tc_ref):
    x = agts_ref[...]
    q = jnp.dot(x, WqT[...], preferred_element_type=jnp.float32)
    q = jnp.maximum(_gn(q, gq[...], bq[...]), 0.0)
    qb = jnp.dot(q, WBT[...], preferred_element_type=jnp.float32)
    a1 = actr_ref[:, 0:1] * w1[...] + actr_ref[:, 1:2] * w2[...]
    ta_ref[...] = _pack16(qb, a1)
    cc = jnp.dot(ctx_ref[...], WCT[...], preferred_element_type=jnp.float32)
    c1 = cctr_ref[:, 0:1] * w1[...] + cctr_ref[:, 1:2] * w2[...]
    tc_ref[...] = _pack16(cc, c1)


def _node_pre(agts, ctx, agt_ctrs, ctx_ctrs, WqT, gq, bq, WBT, WCT,
              w1, w2):
    grid = (N_AGT // RB,)
    row = pl.BlockSpec((RB, D), lambda i: (i, 0))
    ctr = pl.BlockSpec((RB, 2), lambda i: (i, 0))
    full = pl.BlockSpec((D, D), lambda i: (0, 0))
    vec = pl.BlockSpec((1, D), lambda i: (0, 0))
    return pl.pallas_call(
        _node_pre_body,
        grid=grid,
        in_specs=[row, row, ctr, ctr, full, vec, vec, full, full,
                  vec, vec],
        out_specs=[row, row],
        out_shape=[jax.ShapeDtypeStruct((N_AGT, D), jnp.int32),
                   jax.ShapeDtypeStruct((N_CTX, D), jnp.int32)],
    )(agts, ctx, agt_ctrs, ctx_ctrs, WqT, gq, bq, WBT, WCT, w1, w2)


_sc_mesh = plsc.VectorSubcoreMesh(core_axis_name="c", subcore_axis_name="s",
                                  num_cores=NC, num_subcores=NS)


NBUF = 3
NBUF_D = 2
NSEG = 2
E_SEG = E // NSEG
NCHUNK_SEG = E_SEG // CHUNK
NITER = (NCHUNK_SEG + NW - 1) // NW
NITER_PAD = ((NITER + NBUF - 1) // NBUF) * NBUF
NITER_PAD_D = ((NITER + NBUF_D - 1) // NBUF_D) * NBUF_D


@functools.partial(
    pl.kernel,
    out_type=(
        jax.ShapeDtypeStruct((E_SEG, D), jnp.int32),
        jax.ShapeDtypeStruct((E_SEG, D), jnp.int32),
    ),
    mesh=_sc_mesh,
    scratch_types=[
        pltpu.VMEM((NBUF, CHUNK), jnp.int32),
        pltpu.VMEM((NBUF, CHUNK), jnp.int32),
        pltpu.VMEM((NBUF, CHUNK, D), jnp.int32),
        pltpu.VMEM((NBUF, CHUNK, D), jnp.int32),
        pltpu.SemaphoreType.DMA,
        pltpu.SemaphoreType.DMA,
        pltpu.SemaphoreType.DMA,
    ],
)
def _gather_sc(hi_hbm, wi_hbm, ta_hbm, tb_hbm,
               ga_hbm, gx_hbm,
               hi_v, wi_v, arows, xrows, sem0, sem1, sem2):
    c = lax.axis_index("c")
    s = lax.axis_index("s")
    wid = s * NC + c
    sems = (sem0, sem1, sem2)

    def start(k, b):
        cid = wid + NW * k

        @pl.when(cid < NCHUNK_SEG)
        def _():
            off = cid * CHUNK
            pltpu.sync_copy(hi_hbm.at[pl.ds(off, CHUNK)], hi_v.at[b])
            pltpu.sync_copy(wi_hbm.at[pl.ds(off, CHUNK)], wi_v.at[b])
            pltpu.async_copy(ta_hbm.at[hi_v.at[b]], arows.at[b], sems[b])
            pltpu.async_copy(tb_hbm.at[wi_v.at[b]], xrows.at[b], sems[b])

    def drain_and_flush(k, b):
        cid = wid + NW * k

        @pl.when(cid < NCHUNK_SEG)
        def _():
            dummy = ta_hbm.at[pl.ds(0, CHUNK)]
            pltpu.make_async_copy(dummy, arows.at[b], sems[b]).wait()
            pltpu.make_async_copy(dummy, xrows.at[b], sems[b]).wait()
            off = cid * CHUNK
            pltpu.sync_copy(arows.at[b], ga_hbm.at[pl.ds(off, CHUNK)])
            pltpu.sync_copy(xrows.at[b], gx_hbm.at[pl.ds(off, CHUNK)])

    for b in range(NBUF):
        start(b, b)

    @pl.loop(0, NITER_PAD, step=NBUF)
    def outer(k):
        for b in range(NBUF):
            drain_and_flush(k + b, b)
            start(k + b + NBUF, b)


def _edge_mlp_body(ga_ref, gx_ref,
                   bd1, Wd2T, gd2, bd2, AT, gc1, bc1, u_ref):
    qa, a1 = _unpack16(ga_ref[...])
    cc, c1 = _unpack16(gx_ref[...])
    e1 = jnp.maximum(a1 - c1 + bd1[...], 0.0)
    e2 = jnp.dot(e1, Wd2T[...], preferred_element_type=jnp.float32)
    e2 = jnp.maximum(_gn(e2, gd2[...], bd2[...]), 0.0)
    y = (jnp.dot(e2, AT[...], preferred_element_type=jnp.float32)
         + qa + cc)
    u_ref[...] = jnp.maximum(_gn(y, gc1[...], bc1[...]), 0.0)


def _edge_mlp(Ga, Gx, bd1, Wd2T, gd2, bd2, AT, gc1, bc1):
    grid = (E_SEG // BE,)
    row2 = pl.BlockSpec((BE, D), lambda i: (i, 0))
    row = pl.BlockSpec((BE, D), lambda i: (i, 0))
    full = pl.BlockSpec((D, D), lambda i: (0, 0))
    vec = pl.BlockSpec((1, D), lambda i: (0, 0))
    return pl.pallas_call(
        _edge_mlp_body,
        grid=grid,
        in_specs=[row2, row2, vec, full, vec, vec, full, vec, vec],
        out_specs=row,
        out_shape=jax.ShapeDtypeStruct((E_SEG, D), jnp.float32),
    )(Ga, Gx, bd1, Wd2T, gd2, bd2, AT, gc1, bc1)


ZR = 48
RSUB = 624
TAIL = N_AGT - NS * RSUB
CH_PER_CORE = NCHUNK // NC


@functools.partial(
    pl.kernel,
    out_type=jax.ShapeDtypeStruct((NC, N_AGT, D), jnp.float32),
    mesh=_sc_mesh,
    scratch_types=[
        pltpu.VMEM((NBUF_D, CHUNK), jnp.int32),
        pltpu.VMEM((NBUF_D, CHUNK, D), jnp.float32),
        pltpu.VMEM((ZR, D), jnp.float32),
        pltpu.VMEM_SHARED((N_AGT, D), jnp.float32),
        pltpu.SemaphoreType.DMA,
        pltpu.SemaphoreType.DMA,
    ],
)
def _scatter_sc(u_hbm, hi_hbm, p_hbm, hi_v, rows, zbuf, acc_sh,
                sem0, sem1):
    c = lax.axis_index("c")
    s = lax.axis_index("s")
    wid = s * NC + c
    sems = (sem0, sem1)
    zero16 = jnp.zeros((16,), jnp.float32)
    for r in range(ZR):
        for j in range(D // 16):
            zbuf[r, pl.ds(j * 16, 16)] = zero16
    for t in range(RSUB // ZR):
        pltpu.sync_copy(zbuf, acc_sh.at[pl.ds(s * RSUB + t * ZR, ZR)])

    @pl.when(s == NS - 1)
    def _():
        pltpu.sync_copy(zbuf.at[pl.ds(0, TAIL)],
                        acc_sh.at[pl.ds(NS * RSUB, TAIL)])

    plsc.subcore_barrier()

    def start(k, b):
        cid = wid + NW * k

        @pl.when(cid < NCHUNK_SEG)
        def _():
            off = cid * CHUNK
            pltpu.sync_copy(hi_hbm.at[pl.ds(off, CHUNK)], hi_v.at[b])
            pltpu.async_copy(u_hbm.at[pl.ds(off, CHUNK)], rows.at[b],
                             sems[b])

    def drain_and_add(k, b):
        cid = wid + NW * k

        @pl.when(cid < NCHUNK_SEG)
        def _():
            pltpu.make_async_copy(u_hbm.at[pl.ds(0, CHUNK)], rows.at[b],
                                  sems[b]).wait()
            pltpu.sync_copy(rows.at[b], acc_sh.at[hi_v.at[b]], add=True)

    for b in range(NBUF_D):
        start(b, b)

    @pl.loop(0, NITER_PAD_D, step=NBUF_D)
    def outer(k):
        for b in range(NBUF_D):
            drain_and_add(k + b, b)
            start(k + b + NBUF_D, b)

    plsc.subcore_barrier()
    pltpu.sync_copy(acc_sh.at[pl.ds(s * RSUB, RSUB)],
                    p_hbm.at[c, pl.ds(s * RSUB, RSUB)])

    @pl.when(s == NS - 1)
    def _():
        pltpu.sync_copy(acc_sh.at[pl.ds(NS * RSUB, TAIL)],
                        p_hbm.at[c, pl.ds(NS * RSUB, TAIL)])


def _final_body(*refs):
    p_refs = refs[:NSEG]
    (agts_ref, WaT, Wc2T, gn_, bn_, WlT, gl_, bl_, out_ref) = refs[NSEG:]
    x = agts_ref[...]
    u = p_refs[0][0] + p_refs[0][1]
    for pr in p_refs[1:]:
        u = u + pr[0] + pr[1]
    out = (jnp.dot(x, WaT[...], preferred_element_type=jnp.float32)
           + jnp.dot(u, Wc2T[...], preferred_element_type=jnp.float32))
    out = jnp.maximum(_gn(out, gn_[...], bn_[...]), 0.0)
    out = _gn(jnp.dot(out, WlT[...], preferred_element_type=jnp.float32),
              gl_[...], bl_[...])
    out_ref[...] = jnp.maximum(out + x, 0.0)


def _final(parts, agts, WaT, Wc2T, gn_, bn_, WlT, gl_, bl_):
    grid = (N_AGT // RB,)
    row = pl.BlockSpec((RB, D), lambda i: (i, 0))
    prow = pl.BlockSpec((NC, RB, D), lambda i: (0, i, 0))
    full = pl.BlockSpec((D, D), lambda i: (0, 0))
    vec = pl.BlockSpec((1, D), lambda i: (0, 0))
    return pl.pallas_call(
        _final_body,
        grid=grid,
        in_specs=[prow] * NSEG + [row, full, full, vec, vec, full, vec, vec],
        out_specs=row,
        out_shape=jax.ShapeDtypeStruct((N_AGT, D), jnp.float32),
    )(*parts, agts, WaT, Wc2T, gn_, bn_, WlT, gl_, bl_)


def kernel(agts, agt_ctrs, ctx, ctx_ctrs, hi, wi,
           W_d1, b_d1, W_d2, g_d2, b_d2,
           W_q, g_q, b_q,
           W_c1, g_c1, b_c1, W_c2,
           W_a, g_n, b_n,
           W_l, g_l, b_l):
    AT = W_c1[:, :D].T
    BT = W_c1[:, D:2 * D].T
    CT = W_c1[:, 2 * D:].T
    r = lambda v: v.reshape(1, D)

    TA, TB = _node_pre(agts, ctx, agt_ctrs, ctx_ctrs,
                       W_q.T, r(g_q), r(b_q), BT, CT,
                       r(W_d1[:, 0]), r(W_d1[:, 1]))

    parts = []
    for g in range(NSEG):
        hi_s = lax.slice_in_dim(hi, g * E_SEG, (g + 1) * E_SEG)
        wi_s = lax.slice_in_dim(wi, g * E_SEG, (g + 1) * E_SEG)
        Ga, Gx = _gather_sc(hi_s, wi_s, TA, TB)
        u = _edge_mlp(Ga, Gx, r(b_d1),
                      W_d2.T, r(g_d2), r(b_d2), AT, r(g_c1), r(b_c1))
        parts.append(_scatter_sc(u, hi_s))

    return _final(parts, agts, W_a.T, W_c2.T,
                  r(g_n), r(b_n), W_l.T, r(g_l), r(b_l))


# speedup vs baseline: 1.2980x; 1.0027x over previous
import functools

import jax
import jax.numpy as jnp
from jax import lax
from jax.experimental import pallas as pl
from jax.experimental.pallas import tpu as pltpu
from jax.experimental.pallas import tpu_sc as plsc

N_AGT = 10000
N_CTX = 10000
E = 160000
D = 128

NC = 2
NS = 16
NW = NC * NS
CHUNK = 128
NCHUNK = E // CHUNK
RB = 5000
BE = 10000
_EPS = 1e-5


def _gn(x, g, b):
    m = jnp.mean(x, axis=1, keepdims=True)
    xc = x - m
    v = jnp.mean(xc * xc, axis=1, keepdims=True)
    return xc * lax.rsqrt(v + _EPS) * g + b


def _pack16(a, b):
    ab = lax.bitcast_convert_type(a, jnp.int32)
    bb = lax.bitcast_convert_type(b, jnp.int32)
    return jnp.bitwise_or(jnp.bitwise_and(ab, jnp.int32(-65536)),
                          lax.shift_right_logical(bb, 16))


def _unpack16(p):
    a = lax.bitcast_convert_type(
        jnp.bitwise_and(p, jnp.int32(-65536)), jnp.float32)
    b = lax.bitcast_convert_type(lax.shift_left(p, 16), jnp.float32)
    return a, b


def _node_pre_body(agts_ref, ctx_ref, actr_ref, cctr_ref,
                    WqT, gq, bq, WBT, WCT, w1, w2,
                    ta_ref, tc_ref):
    x = agts_ref[...]
    q = jnp.dot(x, WqT[...], preferred_element_type=jnp.float32)
    q = jnp.maximum(_gn(q, gq[...], bq[...]), 0.0)
    qb = jnp.dot(q, WBT[...], preferred_element_type=jnp.float32)
    a1 = actr_ref[:, 0:1] * w1[...] + actr_ref[:, 1:2] * w2[...]
    ta_ref[...] = _pack16(qb, a1)
    cc = jnp.dot(ctx_ref[...], WCT[...], preferred_element_type=jnp.float32)
    c1 = cctr_ref[:, 0:1] * w1[...] + cctr_ref[:, 1:2] * w2[...]
    tc_ref[...] = _pack16(cc, c1)


def _node_pre(agts, ctx, agt_ctrs, ctx_ctrs, WqT, gq, bq, WBT, WCT,
              w1, w2):
    grid = (N_AGT // RB,)
    row = pl.BlockSpec((RB, D), lambda i: (i, 0))
    ctr = pl.BlockSpec((RB, 2), lambda i: (i, 0))
    full = pl.BlockSpec((D, D), lambda i: (0, 0))
    vec = pl.BlockSpec((1, D), lambda i: (0, 0))
    return pl.pallas_call(
        _node_pre_body,
        grid=grid,
        in_specs=[row, row, ctr, ctr, full, vec, vec, full, full,
                  vec, vec],
        out_specs=[row, row],
        out_shape=[jax.ShapeDtypeStruct((N_AGT, D), jnp.int32),
                   jax.ShapeDtypeStruct((N_CTX, D), jnp.int32)],
    )(agts, ctx, agt_ctrs, ctx_ctrs, WqT, gq, bq, WBT, WCT, w1, w2)


_sc_mesh = plsc.VectorSubcoreMesh(core_axis_name="c", subcore_axis_name="s",
                                  num_cores=NC, num_subcores=NS)


NBUF = 3
NBUF_D = 2
NSEG = 2
E_SEG = E // NSEG
NCHUNK_SEG = E_SEG // CHUNK
NITER = (NCHUNK_SEG + NW - 1) // NW
NITER_PAD = ((NITER + NBUF - 1) // NBUF) * NBUF
NITER_PAD_D = ((NITER + NBUF_D - 1) // NBUF_D) * NBUF_D


@functools.partial(
    pl.kernel,
    out_type=(
        jax.ShapeDtypeStruct((E_SEG, D), jnp.int32),
        jax.ShapeDtypeStruct((E_SEG, D), jnp.int32),
    ),
    mesh=_sc_mesh,
    scratch_types=[
        pltpu.VMEM((NBUF, CHUNK), jnp.int32),
        pltpu.VMEM((NBUF, CHUNK), jnp.int32),
        pltpu.VMEM((NBUF, CHUNK, D), jnp.int32),
        pltpu.VMEM((NBUF, CHUNK, D), jnp.int32),
        pltpu.SemaphoreType.DMA,
        pltpu.SemaphoreType.DMA,
        pltpu.SemaphoreType.DMA,
    ],
)
def _gather_sc(hi_hbm, wi_hbm, ta_hbm, tb_hbm,
               ga_hbm, gx_hbm,
               hi_v, wi_v, arows, xrows, sem0, sem1, sem2):
    c = lax.axis_index("c")
    s = lax.axis_index("s")
    wid = s * NC + c
    sems = (sem0, sem1, sem2)

    def start(k, b):
        cid = wid + NW * k

        @pl.when(cid < NCHUNK_SEG)
        def _():
            off = cid * CHUNK
            pltpu.sync_copy(hi_hbm.at[pl.ds(off, CHUNK)], hi_v.at[b])
            pltpu.sync_copy(wi_hbm.at[pl.ds(off, CHUNK)], wi_v.at[b])
            pltpu.async_copy(ta_hbm.at[hi_v.at[b]], arows.at[b], sems[b])
            pltpu.async_copy(tb_hbm.at[wi_v.at[b]], xrows.at[b], sems[b])

    def drain_and_flush(k, b):
        cid = wid + NW * k

        @pl.when(cid < NCHUNK_SEG)
        def _():
            dummy = ta_hbm.at[pl.ds(0, CHUNK)]
            pltpu.make_async_copy(dummy, arows.at[b], sems[b]).wait()
            pltpu.make_async_copy(dummy, xrows.at[b], sems[b]).wait()
            off = cid * CHUNK
            pltpu.sync_copy(arows.at[b], ga_hbm.at[pl.ds(off, CHUNK)])
            pltpu.sync_copy(xrows.at[b], gx_hbm.at[pl.ds(off, CHUNK)])

    for b in range(NBUF):
        start(b, b)

    @pl.loop(0, NITER_PAD, step=NBUF)
    def outer(k):
        for b in range(NBUF):
            drain_and_flush(k + b, b)
            start(k + b + NBUF, b)


def _edge_mlp_body(ga_ref, gx_ref,
                   bd1, Wd2T, gd2, bd2, AT, gc1, bc1, u_ref):
    qa, a1 = _unpack16(ga_ref[...])
    cc, c1 = _unpack16(gx_ref[...])
    e1 = jnp.maximum(a1 - c1 + bd1[...], 0.0)
    e2 = jnp.dot(e1, Wd2T[...], preferred_element_type=jnp.float32)
    e2 = jnp.maximum(_gn(e2, gd2[...], bd2[...]), 0.0)
    y = (jnp.dot(e2, AT[...], preferred_element_type=jnp.float32)
         + qa + cc)
    u_ref[...] = jnp.maximum(_gn(y, gc1[...], bc1[...]), 0.0)


def _edge_mlp(Ga, Gx, bd1, Wd2T, gd2, bd2, AT, gc1, bc1):
    grid = (E_SEG // BE,)
    row2 = pl.BlockSpec((BE, D), lambda i: (i, 0))
    row = pl.BlockSpec((BE, D), lambda i: (i, 0))
    full = pl.BlockSpec((D, D), lambda i: (0, 0))
    vec = pl.BlockSpec((1, D), lambda i: (0, 0))
    return pl.pallas_call(
        _edge_mlp_body,
        grid=grid,
        in_specs=[row2, row2, vec, full, vec, vec, full, vec, vec],
        out_specs=row,
        out_shape=jax.ShapeDtypeStruct((E_SEG, D), jnp.float32),
    )(Ga, Gx, bd1, Wd2T, gd2, bd2, AT, gc1, bc1)


ZR = 48
RSUB = 624
TAIL = N_AGT - NS * RSUB
CH_PER_CORE = NCHUNK // NC


@functools.partial(
    pl.kernel,
    out_type=jax.ShapeDtypeStruct((NC, N_AGT, D), jnp.float32),
    mesh=_sc_mesh,
    scratch_types=[
        pltpu.VMEM((NBUF_D, CHUNK), jnp.int32),
        pltpu.VMEM((NBUF_D, CHUNK, D), jnp.float32),
        pltpu.VMEM((ZR, D), jnp.float32),
        pltpu.VMEM_SHARED((N_AGT, D), jnp.float32),
        pltpu.SemaphoreType.DMA,
        pltpu.SemaphoreType.DMA,
    ],
)
def _scatter_sc(u_hbm, hi_hbm, p_hbm, hi_v, rows, zbuf, acc_sh,
                sem0, sem1):
    c = lax.axis_index("c")
    s = lax.axis_index("s")
    wid = s * NC + c
    sems = (sem0, sem1)
    zero16 = jnp.zeros((16,), jnp.float32)
    for r in range(ZR):
        for j in range(D // 16):
            zbuf[r, pl.ds(j * 16, 16)] = zero16
    for t in range(RSUB // ZR):
        pltpu.sync_copy(zbuf, acc_sh.at[pl.ds(s * RSUB + t * ZR, ZR)])

    @pl.when(s == NS - 1)
    def _():
        pltpu.sync_copy(zbuf.at[pl.ds(0, TAIL)],
                        acc_sh.at[pl.ds(NS * RSUB, TAIL)])

    plsc.subcore_barrier()

    def start(k, b):
        cid = wid + NW * k

        @pl.when(cid < NCHUNK_SEG)
        def _():
            off = cid * CHUNK
            pltpu.sync_copy(hi_hbm.at[pl.ds(off, CHUNK)], hi_v.at[b])
            pltpu.async_copy(u_hbm.at[pl.ds(off, CHUNK)], rows.at[b],
                             sems[b])

    def drain_and_add(k, b):
        cid = wid + NW * k

        @pl.when(cid < NCHUNK_SEG)
        def _():
            pltpu.make_async_copy(u_hbm.at[pl.ds(0, CHUNK)], rows.at[b],
                                  sems[b]).wait()
            pltpu.sync_copy(rows.at[b], acc_sh.at[hi_v.at[b]], add=True)

    for b in range(NBUF_D):
        start(b, b)

    @pl.loop(0, NITER_PAD_D, step=NBUF_D)
    def outer(k):
        for b in range(NBUF_D):
            drain_and_add(k + b, b)
            start(k + b + NBUF_D, b)

    plsc.subcore_barrier()
    pltpu.sync_copy(acc_sh.at[pl.ds(s * RSUB, RSUB)],
                    p_hbm.at[c, pl.ds(s * RSUB, RSUB)])

    @pl.when(s == NS - 1)
    def _():
        pltpu.sync_copy(acc_sh.at[pl.ds(NS * RSUB, TAIL)],
                        p_hbm.at[c, pl.ds(NS * RSUB, TAIL)])


def _final_body(*refs):
    p_refs = refs[:NSEG]
    (agts_ref, WaT, Wc2T, gn_, bn_, WlT, gl_, bl_, out_ref) = refs[NSEG:]
    x = agts_ref[...]
    u = p_refs[0][0] + p_refs[0][1]
    for pr in p_refs[1:]:
        u = u + pr[0] + pr[1]
    out = (jnp.dot(x, WaT[...], preferred_element_type=jnp.float32)
           + jnp.dot(u, Wc2T[...], preferred_element_type=jnp.float32))
    out = jnp.maximum(_gn(out, gn_[...], bn_[...]), 0.0)
    out = _gn(jnp.dot(out, WlT[...], preferred_element_type=jnp.float32),
              gl_[...], bl_[...])
    out_ref[...] = jnp.maximum(out + x, 0.0)


def _final(parts, agts, WaT, Wc2T, gn_, bn_, WlT, gl_, bl_):
    grid = (N_AGT // RB,)
    row = pl.BlockSpec((RB, D), lambda i: (i, 0))
    prow = pl.BlockSpec((NC, RB, D), lambda i: (0, i, 0))
    full = pl.BlockSpec((D, D), lambda i: (0, 0))
    vec = pl.BlockSpec((1, D), lambda i: (0, 0))
    return pl.pallas_call(
        _final_body,
        grid=grid,
        in_specs=[prow] * NSEG + [row, full, full, vec, vec, full, vec, vec],
        out_specs=row,
        out_shape=jax.ShapeDtypeStruct((N_AGT, D), jnp.float32),
    )(*parts, agts, WaT, Wc2T, gn_, bn_, WlT, gl_, bl_)


def kernel(agts, agt_ctrs, ctx, ctx_ctrs, hi, wi,
           W_d1, b_d1, W_d2, g_d2, b_d2,
           W_q, g_q, b_q,
           W_c1, g_c1, b_c1, W_c2,
           W_a, g_n, b_n,
           W_l, g_l, b_l):
    AT = W_c1[:, :D].T
    BT = W_c1[:, D:2 * D].T
    CT = W_c1[:, 2 * D:].T
    r = lambda v: v.reshape(1, D)

    TA, TB = _node_pre(agts, ctx, agt_ctrs, ctx_ctrs,
                       W_q.T, r(g_q), r(b_q), BT, CT,
                       r(W_d1[:, 0]), r(W_d1[:, 1]))

    parts = []
    for g in range(NSEG):
        hi_s = lax.slice_in_dim(hi, g * E_SEG, (g + 1) * E_SEG)
        wi_s = lax.slice_in_dim(wi, g * E_SEG, (g + 1) * E_SEG)
        Ga, Gx = _gather_sc(hi_s, wi_s, TA, TB)
        u = _edge_mlp(Ga, Gx, r(b_d1),
                      W_d2.T, r(g_d2), r(b_d2), AT, r(g_c1), r(b_c1))
        parts.append(_scatter_sc(u, hi_s))

    return _final(parts, agts, W_a.T, W_c2.T,
                  r(g_n), r(b_n), W_l.T, r(g_l), r(b_l))


# speedup vs baseline: 1.3284x; 1.0234x over previous
import functools

import jax
import jax.numpy as jnp
from jax import lax
from jax.experimental import pallas as pl
from jax.experimental.pallas import tpu as pltpu
from jax.experimental.pallas import tpu_sc as plsc

N_AGT = 10000
N_CTX = 10000
E = 160000
D = 128

NC = 2
NS = 16
NW = NC * NS
CHUNK = 128
NCHUNK = E // CHUNK
RB = 5000
BE = 10000
_EPS = 1e-5


def _gn(x, g, b):
    m = jnp.mean(x, axis=1, keepdims=True)
    xc = x - m
    v = jnp.mean(xc * xc, axis=1, keepdims=True)
    return xc * lax.rsqrt(v + _EPS) * g + b


def _pack16(a, b):
    ab = lax.bitcast_convert_type(a, jnp.int32)
    bb = lax.bitcast_convert_type(b, jnp.int32)
    return jnp.bitwise_or(jnp.bitwise_and(ab, jnp.int32(-65536)),
                          lax.shift_right_logical(bb, 16))


def _unpack16(p):
    a = lax.bitcast_convert_type(
        jnp.bitwise_and(p, jnp.int32(-65536)), jnp.float32)
    b = lax.bitcast_convert_type(lax.shift_left(p, 16), jnp.float32)
    return a, b


def _node_pre_body(agts_ref, ctx_ref, actr_ref, cctr_ref,
                    WqT, gq, bq, WBT, WCT, w1, w2,
                    ta_ref, tc_ref):
    x = agts_ref[...]
    q = jnp.dot(x, WqT[...], preferred_element_type=jnp.float32)
    q = jnp.maximum(_gn(q, gq[...], bq[...]), 0.0)
    qb = jnp.dot(q, WBT[...], preferred_element_type=jnp.float32)
    a1 = actr_ref[:, 0:1] * w1[...] + actr_ref[:, 1:2] * w2[...]
    ta_ref[...] = _pack16(qb, a1)
    cc = jnp.dot(ctx_ref[...], WCT[...], preferred_element_type=jnp.float32)
    c1 = cctr_ref[:, 0:1] * w1[...] + cctr_ref[:, 1:2] * w2[...]
    tc_ref[...] = _pack16(cc, c1)


def _node_pre(agts, ctx, agt_ctrs, ctx_ctrs, WqT, gq, bq, WBT, WCT,
              w1, w2):
    grid = (N_AGT // RB,)
    row = pl.BlockSpec((RB, D), lambda i: (i, 0))
    ctr = pl.BlockSpec((RB, 2), lambda i: (i, 0))
    full = pl.BlockSpec((D, D), lambda i: (0, 0))
    vec = pl.BlockSpec((1, D), lambda i: (0, 0))
    return pl.pallas_call(
        _node_pre_body,
        grid=grid,
        in_specs=[row, row, ctr, ctr, full, vec, vec, full, full,
                  vec, vec],
        out_specs=[row, row],
        out_shape=[jax.ShapeDtypeStruct((N_AGT, D), jnp.int32),
                   jax.ShapeDtypeStruct((N_CTX, D), jnp.int32)],
    )(agts, ctx, agt_ctrs, ctx_ctrs, WqT, gq, bq, WBT, WCT, w1, w2)


_sc_mesh = plsc.VectorSubcoreMesh(core_axis_name="c", subcore_axis_name="s",
                                  num_cores=NC, num_subcores=NS)


NBUF = 3
NBUF_D = 2
NSEG = 2
E_SEG = E // NSEG
NCHUNK_SEG = E_SEG // CHUNK
NITER = (NCHUNK_SEG + NW - 1) // NW
NITER_PAD = ((NITER + NBUF - 1) // NBUF) * NBUF
NITER_PAD_D = ((NITER + NBUF_D - 1) // NBUF_D) * NBUF_D


@functools.partial(
    pl.kernel,
    out_type=(
        jax.ShapeDtypeStruct((E_SEG, D), jnp.int32),
        jax.ShapeDtypeStruct((E_SEG, D), jnp.int32),
    ),
    mesh=_sc_mesh,
    scratch_types=[
        pltpu.VMEM((NBUF, 2, CHUNK), jnp.int32),
        pltpu.VMEM((NBUF, CHUNK, D), jnp.int32),
        pltpu.VMEM((NBUF, CHUNK, D), jnp.int32),
        pltpu.SemaphoreType.DMA,
        pltpu.SemaphoreType.DMA,
        pltpu.SemaphoreType.DMA,
    ],
)
def _gather_sc(hiwi_hbm, ta_hbm, tb_hbm,
               ga_hbm, gx_hbm,
               hiwi_v, arows, xrows, sem0, sem1, sem2):
    c = lax.axis_index("c")
    s = lax.axis_index("s")
    wid = s * NC + c
    sems = (sem0, sem1, sem2)

    def start(k, b):
        cid = wid + NW * k

        @pl.when(cid < NCHUNK_SEG)
        def _():
            pltpu.sync_copy(hiwi_hbm.at[cid], hiwi_v.at[b])
            pltpu.async_copy(ta_hbm.at[hiwi_v.at[b, 0]], arows.at[b],
                             sems[b])
            pltpu.async_copy(tb_hbm.at[hiwi_v.at[b, 1]], xrows.at[b],
                             sems[b])

    def drain_and_flush(k, b):
        cid = wid + NW * k

        @pl.when(cid < NCHUNK_SEG)
        def _():
            dummy = ta_hbm.at[pl.ds(0, CHUNK)]
            pltpu.make_async_copy(dummy, arows.at[b], sems[b]).wait()
            pltpu.make_async_copy(dummy, xrows.at[b], sems[b]).wait()
            off = cid * CHUNK
            pltpu.sync_copy(arows.at[b], ga_hbm.at[pl.ds(off, CHUNK)])
            pltpu.sync_copy(xrows.at[b], gx_hbm.at[pl.ds(off, CHUNK)])

    for b in range(NBUF):
        start(b, b)

    @pl.loop(0, NITER_PAD, step=NBUF)
    def outer(k):
        for b in range(NBUF):
            drain_and_flush(k + b, b)
            start(k + b + NBUF, b)


def _edge_mlp_body(ga_ref, gx_ref,
                   bd1, Wd2T, gd2, bd2, AT, gc1, bc1, u_ref):
    qa, a1 = _unpack16(ga_ref[...])
    cc, c1 = _unpack16(gx_ref[...])
    e1 = jnp.maximum(a1 - c1 + bd1[...], 0.0)
    e2 = jnp.dot(e1, Wd2T[...], preferred_element_type=jnp.float32)
    e2 = jnp.maximum(_gn(e2, gd2[...], bd2[...]), 0.0)
    y = (jnp.dot(e2, AT[...], preferred_element_type=jnp.float32)
         + qa + cc)
    u_ref[...] = jnp.maximum(_gn(y, gc1[...], bc1[...]), 0.0)


def _edge_mlp(Ga, Gx, bd1, Wd2T, gd2, bd2, AT, gc1, bc1):
    grid = (E_SEG // BE,)
    row2 = pl.BlockSpec((BE, D), lambda i: (i, 0))
    row = pl.BlockSpec((BE, D), lambda i: (i, 0))
    full = pl.BlockSpec((D, D), lambda i: (0, 0))
    vec = pl.BlockSpec((1, D), lambda i: (0, 0))
    return pl.pallas_call(
        _edge_mlp_body,
        grid=grid,
        in_specs=[row2, row2, vec, full, vec, vec, full, vec, vec],
        out_specs=row,
        out_shape=jax.ShapeDtypeStruct((E_SEG, D), jnp.float32),
    )(Ga, Gx, bd1, Wd2T, gd2, bd2, AT, gc1, bc1)


ZR = 48
RSUB = 624
TAIL = N_AGT - NS * RSUB
CH_PER_CORE = NCHUNK // NC


@functools.partial(
    pl.kernel,
    out_type=jax.ShapeDtypeStruct((NC, N_AGT, D), jnp.float32),
    mesh=_sc_mesh,
    scratch_types=[
        pltpu.VMEM((NBUF_D, CHUNK), jnp.int32),
        pltpu.VMEM((NBUF_D, CHUNK, D), jnp.float32),
        pltpu.VMEM((ZR, D), jnp.float32),
        pltpu.VMEM_SHARED((N_AGT, D), jnp.float32),
        pltpu.SemaphoreType.DMA,
        pltpu.SemaphoreType.DMA,
    ],
)
def _scatter_sc(u_hbm, hi_hbm, p_hbm, hi_v, rows, zbuf, acc_sh,
                sem0, sem1):
    c = lax.axis_index("c")
    s = lax.axis_index("s")
    wid = s * NC + c
    sems = (sem0, sem1)
    zero16 = jnp.zeros((16,), jnp.float32)
    for r in range(ZR):
        for j in range(D // 16):
            zbuf[r, pl.ds(j * 16, 16)] = zero16
    for t in range(RSUB // ZR):
        pltpu.sync_copy(zbuf, acc_sh.at[pl.ds(s * RSUB + t * ZR, ZR)])

    @pl.when(s == NS - 1)
    def _():
        pltpu.sync_copy(zbuf.at[pl.ds(0, TAIL)],
                        acc_sh.at[pl.ds(NS * RSUB, TAIL)])

    plsc.subcore_barrier()

    def start(k, b):
        cid = wid + NW * k

        @pl.when(cid < NCHUNK_SEG)
        def _():
            off = cid * CHUNK
            pltpu.sync_copy(hi_hbm.at[pl.ds(off, CHUNK)], hi_v.at[b])
            pltpu.async_copy(u_hbm.at[pl.ds(off, CHUNK)], rows.at[b],
                             sems[b])

    def drain_and_add(k, b):
        cid = wid + NW * k

        @pl.when(cid < NCHUNK_SEG)
        def _():
            pltpu.make_async_copy(u_hbm.at[pl.ds(0, CHUNK)], rows.at[b],
                                  sems[b]).wait()
            pltpu.sync_copy(rows.at[b], acc_sh.at[hi_v.at[b]], add=True)

    for b in range(NBUF_D):
        start(b, b)

    @pl.loop(0, NITER_PAD_D, step=NBUF_D)
    def outer(k):
        for b in range(NBUF_D):
            drain_and_add(k + b, b)
            start(k + b + NBUF_D, b)

    plsc.subcore_barrier()
    pltpu.sync_copy(acc_sh.at[pl.ds(s * RSUB, RSUB)],
                    p_hbm.at[c, pl.ds(s * RSUB, RSUB)])

    @pl.when(s == NS - 1)
    def _():
        pltpu.sync_copy(acc_sh.at[pl.ds(NS * RSUB, TAIL)],
                        p_hbm.at[c, pl.ds(NS * RSUB, TAIL)])


def _final_body(*refs):
    p_refs = refs[:NSEG]
    (agts_ref, WaT, Wc2T, gn_, bn_, WlT, gl_, bl_, out_ref) = refs[NSEG:]
    x = agts_ref[...]
    u = p_refs[0][0] + p_refs[0][1]
    for pr in p_refs[1:]:
        u = u + pr[0] + pr[1]
    out = (jnp.dot(x, WaT[...], preferred_element_type=jnp.float32)
           + jnp.dot(u, Wc2T[...], preferred_element_type=jnp.float32))
    out = jnp.maximum(_gn(out, gn_[...], bn_[...]), 0.0)
    out = _gn(jnp.dot(out, WlT[...], preferred_element_type=jnp.float32),
              gl_[...], bl_[...])
    out_ref[...] = jnp.maximum(out + x, 0.0)


def _final(parts, agts, WaT, Wc2T, gn_, bn_, WlT, gl_, bl_):
    grid = (N_AGT // RB,)
    row = pl.BlockSpec((RB, D), lambda i: (i, 0))
    prow = pl.BlockSpec((NC, RB, D), lambda i: (0, i, 0))
    full = pl.BlockSpec((D, D), lambda i: (0, 0))
    vec = pl.BlockSpec((1, D), lambda i: (0, 0))
    return pl.pallas_call(
        _final_body,
        grid=grid,
        in_specs=[prow] * NSEG + [row, full, full, vec, vec, full, vec, vec],
        out_specs=row,
        out_shape=jax.ShapeDtypeStruct((N_AGT, D), jnp.float32),
    )(*parts, agts, WaT, Wc2T, gn_, bn_, WlT, gl_, bl_)


def kernel(agts, agt_ctrs, ctx, ctx_ctrs, hi, wi,
           W_d1, b_d1, W_d2, g_d2, b_d2,
           W_q, g_q, b_q,
           W_c1, g_c1, b_c1, W_c2,
           W_a, g_n, b_n,
           W_l, g_l, b_l):
    AT = W_c1[:, :D].T
    BT = W_c1[:, D:2 * D].T
    CT = W_c1[:, 2 * D:].T
    r = lambda v: v.reshape(1, D)

    TA, TB = _node_pre(agts, ctx, agt_ctrs, ctx_ctrs,
                       W_q.T, r(g_q), r(b_q), BT, CT,
                       r(W_d1[:, 0]), r(W_d1[:, 1]))

    parts = []
    for g in range(NSEG):
        hi_s = lax.slice_in_dim(hi, g * E_SEG, (g + 1) * E_SEG)
        wi_s = lax.slice_in_dim(wi, g * E_SEG, (g + 1) * E_SEG)
        hiwi = jnp.stack([hi_s.reshape(NCHUNK_SEG, CHUNK),
                          wi_s.reshape(NCHUNK_SEG, CHUNK)], axis=1)
        Ga, Gx = _gather_sc(hiwi, TA, TB)
        u = _edge_mlp(Ga, Gx, r(b_d1),
                      W_d2.T, r(g_d2), r(b_d2), AT, r(g_c1), r(b_c1))
        parts.append(_scatter_sc(u, hi_s))

    return _final(parts, agts, W_a.T, W_c2.T,
                  r(g_n), r(b_n), W_l.T, r(g_l), r(b_l))


# speedup vs baseline: 1.3296x; 1.0009x over previous
import functools

import jax
import jax.numpy as jnp
from jax import lax
from jax.experimental import pallas as pl
from jax.experimental.pallas import tpu as pltpu
from jax.experimental.pallas import tpu_sc as plsc

N_AGT = 10000
N_CTX = 10000
E = 160000
D = 128

NC = 2
NS = 16
NW = NC * NS
CHUNK = 128
NCHUNK = E // CHUNK
RB = 5000
BE = 10000
_EPS = 1e-5


def _gn(x, g, b):
    m = jnp.mean(x, axis=1, keepdims=True)
    xc = x - m
    v = jnp.mean(xc * xc, axis=1, keepdims=True)
    return xc * lax.rsqrt(v + _EPS) * g + b


def _pack16(a, b):
    ab = lax.bitcast_convert_type(a, jnp.int32)
    bb = lax.bitcast_convert_type(b, jnp.int32)
    return jnp.bitwise_or(jnp.bitwise_and(ab, jnp.int32(-65536)),
                          lax.shift_right_logical(bb, 16))


def _unpack16(p):
    a = lax.bitcast_convert_type(
        jnp.bitwise_and(p, jnp.int32(-65536)), jnp.float32)
    b = lax.bitcast_convert_type(lax.shift_left(p, 16), jnp.float32)
    return a, b


def _node_pre_body(agts_ref, ctx_ref, actr_ref, cctr_ref,
                    WqT, gq, bq, WBT, WCT, w1, w2,
                    ta_ref, tc_ref):
    x = agts_ref[...]
    q = jnp.dot(x, WqT[...], preferred_element_type=jnp.float32)
    q = jnp.maximum(_gn(q, gq[...], bq[...]), 0.0)
    qb = jnp.dot(q, WBT[...], preferred_element_type=jnp.float32)
    a1 = actr_ref[:, 0:1] * w1[...] + actr_ref[:, 1:2] * w2[...]
    ta_ref[...] = _pack16(qb, a1)
    cc = jnp.dot(ctx_ref[...], WCT[...], preferred_element_type=jnp.float32)
    c1 = cctr_ref[:, 0:1] * w1[...] + cctr_ref[:, 1:2] * w2[...]
    tc_ref[...] = _pack16(cc, c1)


def _node_pre(agts, ctx, agt_ctrs, ctx_ctrs, WqT, gq, bq, WBT, WCT,
              w1, w2):
    grid = (N_AGT // RB,)
    row = pl.BlockSpec((RB, D), lambda i: (i, 0))
    ctr = pl.BlockSpec((RB, 2), lambda i: (i, 0))
    full = pl.BlockSpec((D, D), lambda i: (0, 0))
    vec = pl.BlockSpec((1, D), lambda i: (0, 0))
    return pl.pallas_call(
        _node_pre_body,
        grid=grid,
        in_specs=[row, row, ctr, ctr, full, vec, vec, full, full,
                  vec, vec],
        out_specs=[row, row],
        out_shape=[jax.ShapeDtypeStruct((N_AGT, D), jnp.int32),
                   jax.ShapeDtypeStruct((N_CTX, D), jnp.int32)],
    )(agts, ctx, agt_ctrs, ctx_ctrs, WqT, gq, bq, WBT, WCT, w1, w2)


_sc_mesh = plsc.VectorSubcoreMesh(core_axis_name="c", subcore_axis_name="s",
                                  num_cores=NC, num_subcores=NS)


NBUF = 3
NBUF_D = 2
NSEG = 2
E_SEG = E // NSEG
NCHUNK_SEG = E_SEG // CHUNK
NITER = (NCHUNK_SEG + NW - 1) // NW
NITER_PAD = ((NITER + NBUF - 1) // NBUF) * NBUF
NITER_PAD_D = ((NITER + NBUF_D - 1) // NBUF_D) * NBUF_D


@functools.partial(
    pl.kernel,
    out_type=(
        jax.ShapeDtypeStruct((E_SEG, D), jnp.int32),
        jax.ShapeDtypeStruct((E_SEG, D), jnp.int32),
    ),
    mesh=_sc_mesh,
    scratch_types=[
        pltpu.VMEM((NBUF, 2, CHUNK), jnp.int32),
        pltpu.VMEM((NBUF, CHUNK, D), jnp.int32),
        pltpu.VMEM((NBUF, CHUNK, D), jnp.int32),
        pltpu.SemaphoreType.DMA,
        pltpu.SemaphoreType.DMA,
        pltpu.SemaphoreType.DMA,
        pltpu.SemaphoreType.DMA,
        pltpu.SemaphoreType.DMA,
        pltpu.SemaphoreType.DMA,
    ],
)
def _gather_sc(hiwi_hbm, ta_hbm, tb_hbm,
               ga_hbm, gx_hbm,
               hiwi_v, arows, xrows,
               sem0, sem1, sem2, semf0, semf1, semf2):
    c = lax.axis_index("c")
    s = lax.axis_index("s")
    wid = s * NC + c
    sems = (sem0, sem1, sem2)
    semf = (semf0, semf1, semf2)
    nch = (NCHUNK_SEG - wid + NW - 1) // NW

    def drain_flush(b):
        dummy = ta_hbm.at[pl.ds(0, CHUNK)]
        pltpu.make_async_copy(dummy, arows.at[b], semf[b]).wait()
        pltpu.make_async_copy(dummy, xrows.at[b], semf[b]).wait()

    def start(k, b, first):
        cid = wid + NW * k

        @pl.when(cid < NCHUNK_SEG)
        def _():
            if not first:
                drain_flush(b)
            pltpu.sync_copy(hiwi_hbm.at[cid], hiwi_v.at[b])
            pltpu.async_copy(ta_hbm.at[hiwi_v.at[b, 0]], arows.at[b],
                             sems[b])
            pltpu.async_copy(tb_hbm.at[hiwi_v.at[b, 1]], xrows.at[b],
                             sems[b])

    def drain_and_flush(k, b):
        cid = wid + NW * k

        @pl.when(cid < NCHUNK_SEG)
        def _():
            dummy = ta_hbm.at[pl.ds(0, CHUNK)]
            pltpu.make_async_copy(dummy, arows.at[b], sems[b]).wait()
            pltpu.make_async_copy(dummy, xrows.at[b], sems[b]).wait()
            off = cid * CHUNK
            pltpu.async_copy(arows.at[b], ga_hbm.at[pl.ds(off, CHUNK)],
                             semf[b])
            pltpu.async_copy(xrows.at[b], gx_hbm.at[pl.ds(off, CHUNK)],
                             semf[b])

    for b in range(NBUF):
        start(b, b, True)

    @pl.loop(0, NITER_PAD, step=NBUF)
    def outer(k):
        for b in range(NBUF):
            drain_and_flush(k + b, b)
            start(k + b + NBUF, b, False)

    for b in range(NBUF):
        @pl.when(nch > b)
        def _(b=b):
            drain_flush(b)


def _edge_mlp_body(ga_ref, gx_ref,
                   bd1, Wd2T, gd2, bd2, AT, gc1, bc1, u_ref):
    qa, a1 = _unpack16(ga_ref[...])
    cc, c1 = _unpack16(gx_ref[...])
    e1 = jnp.maximum(a1 - c1 + bd1[...], 0.0)
    e2 = jnp.dot(e1, Wd2T[...], preferred_element_type=jnp.float32)
    e2 = jnp.maximum(_gn(e2, gd2[...], bd2[...]), 0.0)
    y = (jnp.dot(e2, AT[...], preferred_element_type=jnp.float32)
         + qa + cc)
    u_ref[...] = jnp.maximum(_gn(y, gc1[...], bc1[...]), 0.0)


def _edge_mlp(Ga, Gx, bd1, Wd2T, gd2, bd2, AT, gc1, bc1):
    grid = (E_SEG // BE,)
    row2 = pl.BlockSpec((BE, D), lambda i: (i, 0))
    row = pl.BlockSpec((BE, D), lambda i: (i, 0))
    full = pl.BlockSpec((D, D), lambda i: (0, 0))
    vec = pl.BlockSpec((1, D), lambda i: (0, 0))
    return pl.pallas_call(
        _edge_mlp_body,
        grid=grid,
        in_specs=[row2, row2, vec, full, vec, vec, full, vec, vec],
        out_specs=row,
        out_shape=jax.ShapeDtypeStruct((E_SEG, D), jnp.float32),
    )(Ga, Gx, bd1, Wd2T, gd2, bd2, AT, gc1, bc1)


ZR = 48
RSUB = 624
TAIL = N_AGT - NS * RSUB
CH_PER_CORE = NCHUNK // NC


@functools.partial(
    pl.kernel,
    out_type=jax.ShapeDtypeStruct((NC, N_AGT, D), jnp.float32),
    mesh=_sc_mesh,
    scratch_types=[
        pltpu.VMEM((NBUF_D, CHUNK), jnp.int32),
        pltpu.VMEM((NBUF_D, CHUNK, D), jnp.float32),
        pltpu.VMEM((ZR, D), jnp.float32),
        pltpu.VMEM_SHARED((N_AGT, D), jnp.float32),
        pltpu.SemaphoreType.DMA,
        pltpu.SemaphoreType.DMA,
    ],
)
def _scatter_sc(u_hbm, hi_hbm, p_hbm, hi_v, rows, zbuf, acc_sh,
                sem0, sem1):
    c = lax.axis_index("c")
    s = lax.axis_index("s")
    wid = s * NC + c
    sems = (sem0, sem1)
    zero16 = jnp.zeros((16,), jnp.float32)
    for r in range(ZR):
        for j in range(D // 16):
            zbuf[r, pl.ds(j * 16, 16)] = zero16
    for t in range(RSUB // ZR):
        pltpu.sync_copy(zbuf, acc_sh.at[pl.ds(s * RSUB + t * ZR, ZR)])

    @pl.when(s == NS - 1)
    def _():
        pltpu.sync_copy(zbuf.at[pl.ds(0, TAIL)],
                        acc_sh.at[pl.ds(NS * RSUB, TAIL)])

    plsc.subcore_barrier()

    def start(k, b):
        cid = wid + NW * k

        @pl.when(cid < NCHUNK_SEG)
        def _():
            off = cid * CHUNK
            pltpu.sync_copy(hi_hbm.at[pl.ds(off, CHUNK)], hi_v.at[b])
            pltpu.async_copy(u_hbm.at[pl.ds(off, CHUNK)], rows.at[b],
                             sems[b])

    def drain_and_add(k, b):
        cid = wid + NW * k

        @pl.when(cid < NCHUNK_SEG)
        def _():
            pltpu.make_async_copy(u_hbm.at[pl.ds(0, CHUNK)], rows.at[b],
                                  sems[b]).wait()
            pltpu.sync_copy(rows.at[b], acc_sh.at[hi_v.at[b]], add=True)

    for b in range(NBUF_D):
        start(b, b)

    @pl.loop(0, NITER_PAD_D, step=NBUF_D)
    def outer(k):
        for b in range(NBUF_D):
            drain_and_add(k + b, b)
            start(k + b + NBUF_D, b)

    plsc.subcore_barrier()
    pltpu.sync_copy(acc_sh.at[pl.ds(s * RSUB, RSUB)],
                    p_hbm.at[c, pl.ds(s * RSUB, RSUB)])

    @pl.when(s == NS - 1)
    def _():
        pltpu.sync_copy(acc_sh.at[pl.ds(NS * RSUB, TAIL)],
                        p_hbm.at[c, pl.ds(NS * RSUB, TAIL)])


def _final_body(*refs):
    p_refs = refs[:NSEG]
    (agts_ref, WaT, Wc2T, gn_, bn_, WlT, gl_, bl_, out_ref) = refs[NSEG:]
    x = agts_ref[...]
    u = p_refs[0][0] + p_refs[0][1]
    for pr in p_refs[1:]:
        u = u + pr[0] + pr[1]
    out = (jnp.dot(x, WaT[...], preferred_element_type=jnp.float32)
           + jnp.dot(u, Wc2T[...], preferred_element_type=jnp.float32))
    out = jnp.maximum(_gn(out, gn_[...], bn_[...]), 0.0)
    out = _gn(jnp.dot(out, WlT[...], preferred_element_type=jnp.float32),
              gl_[...], bl_[...])
    out_ref[...] = jnp.maximum(out + x, 0.0)


def _final(parts, agts, WaT, Wc2T, gn_, bn_, WlT, gl_, bl_):
    grid = (N_AGT // RB,)
    row = pl.BlockSpec((RB, D), lambda i: (i, 0))
    prow = pl.BlockSpec((NC, RB, D), lambda i: (0, i, 0))
    full = pl.BlockSpec((D, D), lambda i: (0, 0))
    vec = pl.BlockSpec((1, D), lambda i: (0, 0))
    return pl.pallas_call(
        _final_body,
        grid=grid,
        in_specs=[prow] * NSEG + [row, full, full, vec, vec, full, vec, vec],
        out_specs=row,
        out_shape=jax.ShapeDtypeStruct((N_AGT, D), jnp.float32),
    )(*parts, agts, WaT, Wc2T, gn_, bn_, WlT, gl_, bl_)


def kernel(agts, agt_ctrs, ctx, ctx_ctrs, hi, wi,
           W_d1, b_d1, W_d2, g_d2, b_d2,
           W_q, g_q, b_q,
           W_c1, g_c1, b_c1, W_c2,
           W_a, g_n, b_n,
           W_l, g_l, b_l):
    AT = W_c1[:, :D].T
    BT = W_c1[:, D:2 * D].T
    CT = W_c1[:, 2 * D:].T
    r = lambda v: v.reshape(1, D)

    TA, TB = _node_pre(agts, ctx, agt_ctrs, ctx_ctrs,
                       W_q.T, r(g_q), r(b_q), BT, CT,
                       r(W_d1[:, 0]), r(W_d1[:, 1]))

    parts = []
    for g in range(NSEG):
        hi_s = lax.slice_in_dim(hi, g * E_SEG, (g + 1) * E_SEG)
        wi_s = lax.slice_in_dim(wi, g * E_SEG, (g + 1) * E_SEG)
        hiwi = jnp.stack([hi_s.reshape(NCHUNK_SEG, CHUNK),
                          wi_s.reshape(NCHUNK_SEG, CHUNK)], axis=1)
        Ga, Gx = _gather_sc(hiwi, TA, TB)
        u = _edge_mlp(Ga, Gx, r(b_d1),
                      W_d2.T, r(g_d2), r(b_d2), AT, r(g_c1), r(b_c1))
        parts.append(_scatter_sc(u, hi_s))

    return _final(parts, agts, W_a.T, W_c2.T,
                  r(g_n), r(b_n), W_l.T, r(g_l), r(b_l))
